# Initial kernel scaffold; baseline (speedup 1.0000x reference)
import jax
import jax.numpy as jnp
from jax.experimental import pallas as pl


def kernel(x, fc1_w, fc1_b, fc2_w, fc2_b, fc3_w, fc3_b, fc4_w, fc4_b, emb):
    raise NotImplementedError("write your pallas kernel here")


# speedup vs baseline: 3.5369x; 3.5369x over previous
import functools

import jax
import jax.numpy as jnp
from jax import lax
from jax.experimental import pallas as pl
from jax.experimental.pallas import tpu as pltpu
from jax.experimental.pallas import tpu_sc as plsc

_NC = 2
_NS = 16
_NW = _NC * _NS
_LANES = 16

_BM = 512

_K = 16
_D = 10
_P = 20


def _enc_body(x_ref, w1_ref, b1_ref, w2_ref, b2_ref, wq_ref, hq_ref,
              h2_ref, sc_ref):
    hi = jax.lax.Precision.HIGHEST
    de = jax.lax.Precision.DEFAULT
    h1 = jnp.maximum(
        jnp.dot(x_ref[...], w1_ref[...], preferred_element_type=jnp.float32,
                precision=de) + b1_ref[...], 0.0)
    h2 = jnp.dot(h1, w2_ref[...], preferred_element_type=jnp.float32,
                 precision=de) + b2_ref[...]
    h2_ref[...] = h2
    sc_ref[...] = jnp.dot(h2, wq_ref[...], preferred_element_type=jnp.float32,
                          precision=hi) - hq_ref[...]


def _dec_body(zq_ref, w3_ref, b3_ref, w4_ref, b4_ref, out_ref):
    hi = jax.lax.Precision.HIGHEST
    h3 = jnp.maximum(
        jnp.dot(zq_ref[...], w3_ref[...], preferred_element_type=jnp.float32,
                precision=hi) + b3_ref[...], 0.0)
    out_ref[...] = jax.nn.sigmoid(
        jnp.dot(h3, w4_ref[...], preferred_element_type=jnp.float32,
                precision=hi) + b4_ref[...])


def _vq_sc_body(rows_w, r_chunk, sc_hbm, embt_hbm, zq_hbm, sc_v, emb_v, out_v):
    cid = lax.axis_index("c")
    sid = lax.axis_index("s")
    wid = sid * _NC + cid
    pltpu.sync_copy(embt_hbm, emb_v)
    lanes = lax.iota(jnp.int32, _LANES)
    n_grp = r_chunk * _P // _LANES

    def grp(g, carry):
        q = g * _LANES + lanes
        q16 = q * _K
        best = plsc.load_gather(sc_v, [q16])
        bidx = jnp.zeros((_LANES,), jnp.int32)
        for k in range(1, _K):
            col = plsc.load_gather(sc_v, [q16 + k])
            m = col > best
            best = jnp.where(m, col, best)
            bidx = jnp.where(m, jnp.full((_LANES,), k, jnp.int32), bidx)
        r = q // _P
        p = q - r * _P
        base = r * 200 + p
        ob = bidx * _D
        for d in range(_D):
            val = plsc.load_gather(emb_v, [ob + d])
            plsc.store_scatter(out_v, [base + _P * d], val)
        return carry

    def chunk(ci, carry):
        row0 = wid * rows_w + ci * r_chunk
        pltpu.sync_copy(sc_hbm.at[pl.ds(row0 * (_P * _K), r_chunk * _P * _K)],
                        sc_v)
        lax.fori_loop(0, n_grp, grp, 0)
        pltpu.sync_copy(out_v, zq_hbm.at[pl.ds(row0 * 200, r_chunk * 200)])
        return carry

    lax.fori_loop(0, rows_w // r_chunk, chunk, 0)


def kernel(x, fc1_w, fc1_b, fc2_w, fc2_b, fc3_w, fc3_b, fc4_w, fc4_b, emb):
    B = x.shape[0]
    f32 = jnp.float32

    w1t = fc1_w.T
    w2t = fc2_w.T
    w3t = fc3_w.T
    w4t = fc4_w.T
    b1 = fc1_b.reshape(1, -1)
    b2 = fc2_b.reshape(1, -1)
    b3 = fc3_b.reshape(1, -1)
    b4 = fc4_b.reshape(1, -1)
    eye_p = jnp.eye(_P, dtype=f32)
    wq = (emb[:, None, None, :] * eye_p[None, :, :, None]).reshape(
        _D * _P, _P * _K)
    hq = jnp.tile(0.5 * jnp.sum(emb * emb, axis=0), _P).reshape(1, _P * _K)
    embt_flat = emb.T.reshape(-1)

    n_bm = B // _BM
    h2, scores = pl.pallas_call(
        _enc_body,
        grid=(n_bm,),
        in_specs=[
            pl.BlockSpec((_BM, 784), lambda i: (i, 0)),
            pl.BlockSpec((784, 400), lambda i: (0, 0)),
            pl.BlockSpec((1, 400), lambda i: (0, 0)),
            pl.BlockSpec((400, 200), lambda i: (0, 0)),
            pl.BlockSpec((1, 200), lambda i: (0, 0)),
            pl.BlockSpec((200, _P * _K), lambda i: (0, 0)),
            pl.BlockSpec((1, _P * _K), lambda i: (0, 0)),
        ],
        out_specs=[
            pl.BlockSpec((_BM, 200), lambda i: (i, 0)),
            pl.BlockSpec((_BM, _P * _K), lambda i: (i, 0)),
        ],
        out_shape=[
            jax.ShapeDtypeStruct((B, 200), f32),
            jax.ShapeDtypeStruct((B, _P * _K), f32),
        ],
    )(x, w1t, b1, w2t, b2, wq, hq)

    rows_w = B // _NW
    r_chunk = 64
    mesh = plsc.VectorSubcoreMesh(core_axis_name="c", subcore_axis_name="s",
                                  num_cores=_NC, num_subcores=_NS)
    zq_flat = pl.kernel(
        functools.partial(_vq_sc_body, rows_w, r_chunk),
        out_type=jax.ShapeDtypeStruct((B * 200,), f32),
        mesh=mesh,
        compiler_params=pltpu.CompilerParams(needs_layout_passes=False),
        scratch_types=[
            pltpu.VMEM((r_chunk * _P * _K,), f32),
            pltpu.VMEM((_K * _D,), f32),
            pltpu.VMEM((r_chunk * 200,), f32),
        ],
    )(scores.reshape(-1), embt_flat)
    z_q = zq_flat.reshape(B, 200)

    recon = pl.pallas_call(
        _dec_body,
        grid=(n_bm,),
        in_specs=[
            pl.BlockSpec((_BM, 200), lambda i: (i, 0)),
            pl.BlockSpec((200, 400), lambda i: (0, 0)),
            pl.BlockSpec((1, 400), lambda i: (0, 0)),
            pl.BlockSpec((400, 784), lambda i: (0, 0)),
            pl.BlockSpec((1, 784), lambda i: (0, 0)),
        ],
        out_specs=pl.BlockSpec((_BM, 784), lambda i: (i, 0)),
        out_shape=jax.ShapeDtypeStruct((B, 784), f32),
    )(z_q, w3t, b3, w4t, b4)

    z_e = h2.reshape(B, _D, _P)
    return recon, z_e, z_q


# speedup vs baseline: 4.4209x; 1.2500x over previous
import functools

import jax
import jax.numpy as jnp
from jax import lax
from jax.experimental import pallas as pl
from jax.experimental.pallas import tpu as pltpu
from jax.experimental.pallas import tpu_sc as plsc

_NC = 2
_NS = 16
_NW = _NC * _NS
_LANES = 16

_BM = 512

_K = 16
_D = 10
_P = 20


def _enc_body(x_ref, w1_ref, b1_ref, w2_ref, b2_ref, wq_ref, hq_ref,
              h2_ref, sc_ref):
    hi = jax.lax.Precision.HIGHEST
    de = jax.lax.Precision.DEFAULT
    h1 = jnp.maximum(
        jnp.dot(x_ref[...], w1_ref[...], preferred_element_type=jnp.float32,
                precision=de) + b1_ref[...], 0.0)
    h2 = jnp.dot(h1, w2_ref[...], preferred_element_type=jnp.float32,
                 precision=de) + b2_ref[...]
    h2_ref[...] = h2
    sc_ref[...] = jnp.dot(h2, wq_ref[...], preferred_element_type=jnp.float32,
                          precision=hi) - hq_ref[...]


def _dec_body(zq_ref, w3_ref, b3_ref, w4_ref, b4_ref, out_ref):
    de = jax.lax.Precision.DEFAULT
    h3 = jnp.maximum(
        jnp.dot(zq_ref[...], w3_ref[...], preferred_element_type=jnp.float32,
                precision=de) + b3_ref[...], 0.0)
    out_ref[...] = jax.nn.sigmoid(
        jnp.dot(h3, w4_ref[...], preferred_element_type=jnp.float32,
                precision=de) + b4_ref[...])


def _vq_sc_body(rows_w, r_chunk, sc_hbm, embt_hbm, zq_hbm, sc_v, emb_v, out_v):
    cid = lax.axis_index("c")
    sid = lax.axis_index("s")
    wid = sid * _NC + cid
    pltpu.sync_copy(embt_hbm, emb_v)
    lanes = lax.iota(jnp.int32, _LANES)
    n_grp = r_chunk * _P // _LANES

    def grp(g, carry):
        q = g * _LANES + lanes
        q16 = q * _K
        best = plsc.load_gather(sc_v, [q16])
        bidx = jnp.zeros((_LANES,), jnp.int32)
        for k in range(1, _K):
            col = plsc.load_gather(sc_v, [q16 + k])
            m = col > best
            best = jnp.where(m, col, best)
            bidx = jnp.where(m, jnp.full((_LANES,), k, jnp.int32), bidx)
        r = q // _P
        p = q - r * _P
        base = r * 200 + p
        ob = bidx * _D
        for d in range(_D):
            val = plsc.load_gather(emb_v, [ob + d])
            plsc.store_scatter(out_v, [base + _P * d], val)
        return carry

    def chunk(ci, carry):
        row0 = wid * rows_w + ci * r_chunk
        pltpu.sync_copy(sc_hbm.at[pl.ds(row0 * (_P * _K), r_chunk * _P * _K)],
                        sc_v)
        lax.fori_loop(0, n_grp, grp, 0)
        pltpu.sync_copy(out_v, zq_hbm.at[pl.ds(row0 * 200, r_chunk * 200)])
        return carry

    lax.fori_loop(0, rows_w // r_chunk, chunk, 0)


def kernel(x, fc1_w, fc1_b, fc2_w, fc2_b, fc3_w, fc3_b, fc4_w, fc4_b, emb):
    B = x.shape[0]
    f32 = jnp.float32

    w1t = fc1_w.T
    w2t = fc2_w.T
    w3t = fc3_w.T
    w4t = fc4_w.T
    b1 = fc1_b.reshape(1, -1)
    b2 = fc2_b.reshape(1, -1)
    b3 = fc3_b.reshape(1, -1)
    b4 = fc4_b.reshape(1, -1)
    eye_p = jnp.eye(_P, dtype=f32)
    wq = (emb[:, None, None, :] * eye_p[None, :, :, None]).reshape(
        _D * _P, _P * _K)
    hq = jnp.tile(0.5 * jnp.sum(emb * emb, axis=0), _P).reshape(1, _P * _K)
    embt_flat = emb.T.reshape(-1)

    n_bm = B // _BM
    h2, scores = pl.pallas_call(
        _enc_body,
        grid=(n_bm,),
        in_specs=[
            pl.BlockSpec((_BM, 784), lambda i: (i, 0)),
            pl.BlockSpec((784, 400), lambda i: (0, 0)),
            pl.BlockSpec((1, 400), lambda i: (0, 0)),
            pl.BlockSpec((400, 200), lambda i: (0, 0)),
            pl.BlockSpec((1, 200), lambda i: (0, 0)),
            pl.BlockSpec((200, _P * _K), lambda i: (0, 0)),
            pl.BlockSpec((1, _P * _K), lambda i: (0, 0)),
        ],
        out_specs=[
            pl.BlockSpec((_BM, 200), lambda i: (i, 0)),
            pl.BlockSpec((_BM, _P * _K), lambda i: (i, 0)),
        ],
        out_shape=[
            jax.ShapeDtypeStruct((B, 200), f32),
            jax.ShapeDtypeStruct((B, _P * _K), f32),
        ],
    )(x, w1t, b1, w2t, b2, wq, hq)

    rows_w = B // _NW
    r_chunk = 64
    mesh = plsc.VectorSubcoreMesh(core_axis_name="c", subcore_axis_name="s",
                                  num_cores=_NC, num_subcores=_NS)
    zq_flat = pl.kernel(
        functools.partial(_vq_sc_body, rows_w, r_chunk),
        out_type=jax.ShapeDtypeStruct((B * 200,), f32),
        mesh=mesh,
        compiler_params=pltpu.CompilerParams(needs_layout_passes=False),
        scratch_types=[
            pltpu.VMEM((r_chunk * _P * _K,), f32),
            pltpu.VMEM((_K * _D,), f32),
            pltpu.VMEM((r_chunk * 200,), f32),
        ],
    )(scores.reshape(-1), embt_flat)
    z_q = zq_flat.reshape(B, 200)

    recon = pl.pallas_call(
        _dec_body,
        grid=(n_bm,),
        in_specs=[
            pl.BlockSpec((_BM, 200), lambda i: (i, 0)),
            pl.BlockSpec((200, 400), lambda i: (0, 0)),
            pl.BlockSpec((1, 400), lambda i: (0, 0)),
            pl.BlockSpec((400, 784), lambda i: (0, 0)),
            pl.BlockSpec((1, 784), lambda i: (0, 0)),
        ],
        out_specs=pl.BlockSpec((_BM, 784), lambda i: (i, 0)),
        out_shape=jax.ShapeDtypeStruct((B, 784), f32),
    )(z_q, w3t, b3, w4t, b4)

    z_e = h2.reshape(B, _D, _P)
    return recon, z_e, z_q


# speedup vs baseline: 9.1656x; 2.0732x over previous
import functools

import jax
import jax.numpy as jnp
from jax import lax
from jax.experimental import pallas as pl
from jax.experimental.pallas import tpu as pltpu
from jax.experimental.pallas import tpu_sc as plsc

_NC = 2
_NS = 16
_NW = _NC * _NS
_LANES = 16

_BM = 512

_K = 16
_D = 10
_P = 20


def _enc_body(xt_ref, w1_ref, b1_ref, w2_ref, b2_ref, wq_ref, hq_ref,
              h2t_ref, sct_ref):
    hi = jax.lax.Precision.HIGHEST
    de = jax.lax.Precision.DEFAULT
    h1t = jnp.maximum(
        jnp.dot(w1_ref[...], xt_ref[...], preferred_element_type=jnp.float32,
                precision=de) + b1_ref[...], 0.0)
    h2t = jnp.dot(w2_ref[...], h1t, preferred_element_type=jnp.float32,
                  precision=de) + b2_ref[...]
    h2t_ref[...] = h2t
    sct_ref[0] = jnp.dot(wq_ref[...], h2t, preferred_element_type=jnp.float32,
                         precision=hi) - hq_ref[...]


def _dec_body(zqc_ref, w3_ref, b3_ref, w4_ref, b4_ref, rt_ref, zqt_ref):
    de = jax.lax.Precision.DEFAULT
    zqt = zqc_ref[0]
    h3t = jnp.maximum(
        jnp.dot(w3_ref[...], zqt, preferred_element_type=jnp.float32,
                precision=de) + b3_ref[...], 0.0)
    rt_ref[...] = jax.nn.sigmoid(
        jnp.dot(w4_ref[...], h3t, preferred_element_type=jnp.float32,
                precision=de) + b4_ref[...])
    zqt_ref[...] = zqt


def _vq_sc_body(sct_hbm, embt_hbm, zqc_hbm, in_v, emb_v, out_v):
    cid = lax.axis_index("c")
    sid = lax.axis_index("s")
    wid = sid * _NC + cid
    pltpu.sync_copy(embt_hbm, emb_v)
    n_grp = _BM // _LANES

    def grp(p, g, _):
        b0 = g * _LANES
        best = in_v[0, pl.ds(b0, _LANES)]
        bidx = jnp.zeros((_LANES,), jnp.int32)
        for k in range(1, _K):
            col = in_v[k, pl.ds(b0, _LANES)]
            m = col > best
            best = jnp.where(m, col, best)
            bidx = jnp.where(m, jnp.full((_LANES,), k, jnp.int32), bidx)
        ob = bidx * _D
        for d in range(_D):
            val = plsc.load_gather(emb_v, [ob + d])
            out_v[_P * d + p, pl.ds(b0, _LANES)] = val
        return _

    def pchunk(p, carry):
        pltpu.sync_copy(sct_hbm.at[wid, pl.ds(_K * p, _K), :], in_v)
        lax.fori_loop(0, n_grp, functools.partial(grp, p), 0)
        return carry

    lax.fori_loop(0, _P, pchunk, 0)
    pltpu.sync_copy(out_v, zqc_hbm.at[wid])


def kernel(x, fc1_w, fc1_b, fc2_w, fc2_b, fc3_w, fc3_b, fc4_w, fc4_b, emb):
    B = x.shape[0]
    f32 = jnp.float32
    n_bm = B // _BM

    xt = x.T
    b1c = fc1_b.reshape(-1, 1)
    b2c = fc2_b.reshape(-1, 1)
    b3c = fc3_b.reshape(-1, 1)
    b4c = fc4_b.reshape(-1, 1)
    eye_p = jnp.eye(_P, dtype=f32)
    wqt = (emb.T[None, :, :, None] * eye_p[:, None, None, :]).reshape(
        _P * _K, _D * _P)
    hqc = jnp.tile(0.5 * jnp.sum(emb * emb, axis=0), _P).reshape(_P * _K, 1)
    embt_flat = emb.T.reshape(-1)

    h2t, sct = pl.pallas_call(
        _enc_body,
        grid=(n_bm,),
        in_specs=[
            pl.BlockSpec((784, _BM), lambda i: (0, i)),
            pl.BlockSpec((400, 784), lambda i: (0, 0)),
            pl.BlockSpec((400, 1), lambda i: (0, 0)),
            pl.BlockSpec((200, 400), lambda i: (0, 0)),
            pl.BlockSpec((200, 1), lambda i: (0, 0)),
            pl.BlockSpec((_P * _K, 200), lambda i: (0, 0)),
            pl.BlockSpec((_P * _K, 1), lambda i: (0, 0)),
        ],
        out_specs=[
            pl.BlockSpec((200, _BM), lambda i: (0, i)),
            pl.BlockSpec((1, _P * _K, _BM), lambda i: (i, 0, 0)),
        ],
        out_shape=[
            jax.ShapeDtypeStruct((200, B), f32),
            jax.ShapeDtypeStruct((n_bm, _P * _K, _BM), f32),
        ],
    )(xt, fc1_w, b1c, fc2_w, b2c, wqt, hqc)

    mesh = plsc.VectorSubcoreMesh(core_axis_name="c", subcore_axis_name="s",
                                  num_cores=_NC, num_subcores=_NS)
    zqc = pl.kernel(
        _vq_sc_body,
        out_type=jax.ShapeDtypeStruct((n_bm, _D * _P, _BM), f32),
        mesh=mesh,
        compiler_params=pltpu.CompilerParams(needs_layout_passes=False),
        scratch_types=[
            pltpu.VMEM((_K, _BM), f32),
            pltpu.VMEM((_K * _D,), f32),
            pltpu.VMEM((_D * _P, _BM), f32),
        ],
    )(sct, embt_flat)

    recont, zqt = pl.pallas_call(
        _dec_body,
        grid=(n_bm,),
        in_specs=[
            pl.BlockSpec((1, 200, _BM), lambda i: (i, 0, 0)),
            pl.BlockSpec((400, 200), lambda i: (0, 0)),
            pl.BlockSpec((400, 1), lambda i: (0, 0)),
            pl.BlockSpec((784, 400), lambda i: (0, 0)),
            pl.BlockSpec((784, 1), lambda i: (0, 0)),
        ],
        out_specs=[
            pl.BlockSpec((784, _BM), lambda i: (0, i)),
            pl.BlockSpec((200, _BM), lambda i: (0, i)),
        ],
        out_shape=[
            jax.ShapeDtypeStruct((784, B), f32),
            jax.ShapeDtypeStruct((200, B), f32),
        ],
    )(zqc, fc3_w, b3c, fc4_w, b4c)

    recon = recont.T
    z_q = zqt.T
    z_e = h2t.reshape(_D, _P, B).transpose(2, 0, 1)
    return recon, z_e, z_q
